# Initial kernel scaffold; baseline (speedup 1.0000x reference)
import jax
import jax.numpy as jnp
from jax.experimental import pallas as pl


def kernel(x, mf_indices):
    raise NotImplementedError("write your pallas kernel here")


# speedup vs baseline: 8158.5167x; 8158.5167x over previous
import jax
import jax.numpy as jnp
from jax.experimental import pallas as pl

_B = 4096
_NV = 7
_NM = 3
_R = 2187
_TB = 512


def _body(x_ref, idx_ref, out_ref):
    acc = None
    for v in range(_NV):
        iv = idx_ref[v : v + 1, :]
        x0 = x_ref[:, 3 * v : 3 * v + 1]
        x1 = x_ref[:, 3 * v + 1 : 3 * v + 2]
        x2 = x_ref[:, 3 * v + 2 : 3 * v + 3]
        sel = jnp.where(iv == 0, x0, jnp.where(iv == 1, x1, x2))
        acc = sel if acc is None else acc * sel
    out_ref[...] = acc


def kernel(x, mf_indices):
    B = x.shape[0]
    xf = x.reshape(B, _NV * _NM)
    idx_t = jnp.pad(mf_indices.T, ((0, 1), (0, 0)))
    grid = (B // _TB,)
    return pl.pallas_call(
        _body,
        grid=grid,
        in_specs=[
            pl.BlockSpec((_TB, _NV * _NM), lambda i: (i, 0)),
            pl.BlockSpec((8, _R), lambda i: (0, 0)),
        ],
        out_specs=pl.BlockSpec((_TB, _R), lambda i: (i, 0)),
        out_shape=jax.ShapeDtypeStruct((B, _R), jnp.float32),
    )(xf, idx_t)
